# baseline (speedup 1.0000x reference)
import jax
import jax.numpy as jnp
from jax import lax
from jax.experimental import pallas as pl
from jax.experimental.pallas import tpu as pltpu
from jax.experimental.pallas import tpu_sc as plsc

VOCAB = 100000
EMBED = 128
BATCH = 4096
HIST = 50

NC = 2
NS = 16
NW = NC * NS

PARTS = 4
B_PART = BATCH // PARTS
B_PER_W = B_PART // NW
G = 2
CHUNKS = B_PER_W // G
RING = 8
H = RING // 2


def _gather_body(table_hbm, idx_hbm, out_hbm, idx_v, *ring):
    bufs = ring[:RING]
    gsems = ring[RING:2 * RING]
    wsems0 = ring[2 * RING:3 * RING]
    wsems1 = ring[3 * RING:]

    wid = lax.axis_index("s") * NC + lax.axis_index("c")
    batch_base = wid * B_PER_W

    pltpu.sync_copy(idx_hbm.at[wid], idx_v)

    def fire_gather(r, j):
        pltpu.async_copy(table_hbm.at[idx_v.at[j]], bufs[r], gsems[r])

    def wait_gather(r, j):
        pltpu.make_async_copy(table_hbm.at[idx_v.at[j]], bufs[r],
                              gsems[r]).wait()

    def fire_write(r, j):
        pltpu.async_copy(bufs[r].at[pl.ds(0, HIST)],
                         out_hbm.at[batch_base + j * G], wsems0[r])
        pltpu.async_copy(bufs[r].at[pl.ds(HIST, HIST)],
                         out_hbm.at[batch_base + j * G + 1], wsems1[r])

    def wait_write(r, j):
        pltpu.make_async_copy(bufs[r].at[pl.ds(0, HIST)],
                              out_hbm.at[batch_base + j * G],
                              wsems0[r]).wait()
        pltpu.make_async_copy(bufs[r].at[pl.ds(HIST, HIST)],
                              out_hbm.at[batch_base + j * G + 1],
                              wsems1[r]).wait()

    def retire(j, r):
        wait_gather(r, j)
        fire_write(r, j)
        wait_write((r + H) % RING, j - H)
        fire_gather((r + H) % RING, j + H)

    for k in range(min(RING, CHUNKS)):
        fire_gather(k, k)
    for j in range(H):
        wait_gather(j, j)
        fire_write(j, j)

    end = CHUNKS - H
    loop_start = min(-(-H // RING) * RING, end)
    for j in range(H, loop_start):
        retire(j, j % RING)

    n_iter = (end - loop_start) // RING

    def step(m, carry):
        j0 = loop_start + m * RING
        for r in range(RING):
            retire(j0 + r, r)
        return carry

    if n_iter > 0:
        lax.fori_loop(0, n_iter, step, 0)

    for j in range(loop_start + n_iter * RING, end):
        retire(j, j % RING)

    for j in range(max(end, H), CHUNKS):
        wait_gather(j % RING, j)
        fire_write(j % RING, j)
        wait_write((j + H) % RING, j - H)
    for j in range(max(end, H), CHUNKS):
        wait_write(j % RING, j)


@jax.jit
def _embed(indices, table):
    mesh = plsc.VectorSubcoreMesh(
        core_axis_name="c", subcore_axis_name="s", num_cores=NC, num_subcores=NS
    )
    idx4 = indices.reshape(PARTS, NW, CHUNKS, G * HIST)
    sc_part = pl.kernel(
        _gather_body,
        out_type=jax.ShapeDtypeStruct((B_PART, HIST, EMBED), jnp.float32),
        mesh=mesh,
        scratch_types=(
            [pltpu.VMEM((CHUNKS, G * HIST), jnp.int32)]
            + [pltpu.VMEM((G * HIST, EMBED), jnp.float32) for _ in range(RING)]
            + [pltpu.SemaphoreType.DMA for _ in range(3 * RING)]
        ),
    )
    parts = [sc_part(table, idx4[s]) for s in range(PARTS)]
    return jnp.concatenate(parts, axis=0)


def kernel(indices, table):
    return _embed(indices, table)


# speedup vs baseline: 1.0142x; 1.0142x over previous
import functools

import jax
import jax.numpy as jnp
from jax import lax
from jax.experimental import pallas as pl
from jax.experimental.pallas import tpu as pltpu
from jax.experimental.pallas import tpu_sc as plsc

VOCAB = 100000
EMBED = 128
BATCH = 4096
HIST = 50

NC = 2
NS = 16
NW = NC * NS

PARTS = 4
B_PART = BATCH // PARTS
B_PER_W = B_PART // NW
WB = 8
SLABS = B_PER_W // WB
SLAB_ROWS = WB * HIST

ASM_GRID = 16
ASM_B = B_PART // ASM_GRID


def _gather_body(part, table_hbm, idx_hbm, out_hbm, idx_v, buf_a, buf_b,
                 *sems):
    bufs = (buf_a, buf_b)
    gsems = (sems[:WB], sems[WB:2 * WB])
    wsems = sems[2 * WB:]

    wid = lax.axis_index("s") * NC + lax.axis_index("c")
    batch_base = part * B_PART + wid * B_PER_W
    row_base = wid * B_PER_W * HIST

    pltpu.sync_copy(idx_hbm.at[pl.ds(batch_base, B_PER_W)], idx_v)

    def fire_gathers(p, sb):
        for q in range(WB):
            pltpu.async_copy(table_hbm.at[idx_v.at[sb * WB + q]],
                             bufs[p].at[pl.ds(q * HIST, HIST)], gsems[p][q])

    def wait_gathers(p, sb):
        for q in range(WB):
            pltpu.make_async_copy(table_hbm.at[idx_v.at[sb * WB + q]],
                                  bufs[p].at[pl.ds(q * HIST, HIST)],
                                  gsems[p][q]).wait()

    def fire_write(p, sb):
        pltpu.async_copy(
            bufs[p], out_hbm.at[pl.ds(row_base + sb * SLAB_ROWS, SLAB_ROWS)],
            wsems[p])

    def wait_write(p, sb):
        pltpu.make_async_copy(
            bufs[p], out_hbm.at[pl.ds(row_base + sb * SLAB_ROWS, SLAB_ROWS)],
            wsems[p]).wait()

    fire_gathers(0, 0)
    fire_gathers(1, 1)
    wait_gathers(0, 0)
    fire_write(0, 0)
    wait_gathers(1, 1)
    fire_write(1, 1)

    def step(m, carry):
        sb = 2 * m + 2
        wait_write(0, sb - 2)
        fire_gathers(0, sb)
        wait_write(1, sb - 1)
        fire_gathers(1, sb + 1)
        wait_gathers(0, sb)
        fire_write(0, sb)
        wait_gathers(1, sb + 1)
        fire_write(1, sb + 1)
        return carry

    lax.fori_loop(0, SLABS // 2 - 1, step, 0)

    wait_write(0, SLABS - 2)
    wait_write(1, SLABS - 1)


def _asm_body(acc_ref, part_ref, out_ref):
    out_ref[...] = part_ref[...].reshape(ASM_B, HIST, EMBED)


@jax.jit
def _embed(indices, table):
    mesh = plsc.VectorSubcoreMesh(
        core_axis_name="c", subcore_axis_name="s", num_cores=NC, num_subcores=NS
    )
    sc_scratch = (
        [pltpu.VMEM((B_PER_W, HIST), jnp.int32)]
        + [pltpu.VMEM((SLAB_ROWS, EMBED), jnp.float32) for _ in range(2)]
        + [pltpu.SemaphoreType.DMA for _ in range(2 * WB + 2)]
    )
    acc = jnp.zeros((BATCH, HIST, EMBED), jnp.float32)
    for part in range(PARTS):
        flat = pl.kernel(
            functools.partial(_gather_body, part),
            out_type=jax.ShapeDtypeStruct((B_PART * HIST, EMBED), jnp.float32),
            mesh=mesh,
            scratch_types=list(sc_scratch),
        )(table, indices)
        acc = pl.pallas_call(
            _asm_body,
            out_shape=jax.ShapeDtypeStruct((BATCH, HIST, EMBED), jnp.float32),
            grid=(ASM_GRID,),
            in_specs=[
                pl.BlockSpec((1, HIST, EMBED), lambda g: (0, 0, 0)),
                pl.BlockSpec((ASM_B * HIST, EMBED), lambda g: (g, 0)),
            ],
            out_specs=pl.BlockSpec(
                (ASM_B, HIST, EMBED),
                functools.partial(
                    lambda p, g: (p * ASM_GRID + g, 0, 0), part)),
            input_output_aliases={0: 0},
        )(acc, flat)
    return acc


def kernel(indices, table):
    return _embed(indices, table)


# speedup vs baseline: 1.8250x; 1.7994x over previous
import jax
import jax.numpy as jnp
from jax import lax
from jax.experimental import pallas as pl
from jax.experimental.pallas import tpu as pltpu
from jax.experimental.pallas import tpu_sc as plsc

VOCAB = 100000
EMBED = 128
BATCH = 4096
HIST = 50

NC = 2
NS = 16
NW = NC * NS

B_PER_W = BATCH // NW
G = 2
CHUNKS = B_PER_W // G
RING = 8
H = RING // 2


def _gather_body(table_hbm, idx_hbm, out_hbm, idx_v, *ring):
    bufs = ring[:RING]
    gsems = ring[RING:2 * RING]
    wsems0 = ring[2 * RING:3 * RING]
    wsems1 = ring[3 * RING:]

    wid = lax.axis_index("s") * NC + lax.axis_index("c")
    batch_base = wid * B_PER_W

    pltpu.sync_copy(idx_hbm.at[wid], idx_v)

    def fire_gather(r, j):
        pltpu.async_copy(table_hbm.at[idx_v.at[j]], bufs[r], gsems[r])

    def wait_gather(r, j):
        pltpu.make_async_copy(table_hbm.at[idx_v.at[j]], bufs[r],
                              gsems[r]).wait()

    def fire_write(r, j):
        pltpu.async_copy(bufs[r].at[pl.ds(0, HIST)],
                         out_hbm.at[batch_base + j * G], wsems0[r])
        pltpu.async_copy(bufs[r].at[pl.ds(HIST, HIST)],
                         out_hbm.at[batch_base + j * G + 1], wsems1[r])

    def wait_write(r, j):
        pltpu.make_async_copy(bufs[r].at[pl.ds(0, HIST)],
                              out_hbm.at[batch_base + j * G],
                              wsems0[r]).wait()
        pltpu.make_async_copy(bufs[r].at[pl.ds(HIST, HIST)],
                              out_hbm.at[batch_base + j * G + 1],
                              wsems1[r]).wait()

    def retire(j, r):
        wait_gather(r, j)
        fire_write(r, j)
        wait_write((r + H) % RING, j - H)
        fire_gather((r + H) % RING, j + H)

    for k in range(RING):
        fire_gather(k, k)
    for j in range(H):
        wait_gather(j, j)
        fire_write(j, j)

    end = CHUNKS - H
    loop_start = min(-(-H // RING) * RING, end)
    for j in range(H, loop_start):
        retire(j, j % RING)

    n_iter = (end - loop_start) // RING

    def step(m, carry):
        j0 = loop_start + m * RING
        for r in range(RING):
            retire(j0 + r, r)
        return carry

    if n_iter > 0:
        lax.fori_loop(0, n_iter, step, 0)

    for j in range(loop_start + n_iter * RING, end):
        retire(j, j % RING)

    for j in range(max(end, H), CHUNKS):
        wait_gather(j % RING, j)
        fire_write(j % RING, j)
        wait_write((j + H) % RING, j - H)
    for j in range(max(end, H), CHUNKS):
        wait_write(j % RING, j)


@jax.jit
def _embed(indices, table):
    mesh = plsc.VectorSubcoreMesh(
        core_axis_name="c", subcore_axis_name="s", num_cores=NC, num_subcores=NS
    )
    idx3 = indices.reshape(NW, CHUNKS, G * HIST)
    return pl.kernel(
        _gather_body,
        out_type=jax.ShapeDtypeStruct((BATCH, HIST, EMBED), jnp.float32),
        mesh=mesh,
        scratch_types=(
            [pltpu.VMEM((CHUNKS, G * HIST), jnp.int32)]
            + [pltpu.VMEM((G * HIST, EMBED), jnp.float32) for _ in range(RING)]
            + [pltpu.SemaphoreType.DMA for _ in range(3 * RING)]
        ),
    )(table, idx3)


def kernel(indices, table):
    return _embed(indices, table)
